# baseline (speedup 1.0000x reference)
import functools

import jax
import jax.numpy as jnp
from jax import lax
from jax.experimental import pallas as pl
from jax.experimental.pallas import tpu as pltpu
from jax.experimental.pallas import tpu_sc as plsc

GLOVE_DIM = 300
D_MODEL = 768
DIM_PAD = 384
N_CHUNKS = 5


def _make_sc_gather(num_rows: int, row_off: int):
    info = plsc.get_sparse_core_info()
    nc, ns = info.num_cores, info.num_subcores
    nw = nc * ns
    assert num_rows % (8 * nw) == 0
    b_per_w = num_rows // nw
    chunk = 80
    assert b_per_w % chunk == 0 and chunk % 8 == 0
    n_chunks = b_per_w // chunk
    n_slices = DIM_PAD // 128

    mesh = plsc.VectorSubcoreMesh(core_axis_name="c", subcore_axis_name="s")

    @functools.partial(
        pl.kernel,
        mesh=mesh,
        out_type=jax.ShapeDtypeStruct((num_rows, DIM_PAD), jnp.float32),
        scratch_types=[
            pltpu.VMEM((2, chunk), jnp.int32),
            pltpu.VMEM((2, chunk, DIM_PAD), jnp.float32),
            pltpu.SemaphoreType.DMA,
            pltpu.SemaphoreType.DMA,
        ],
    )
    def gather(tab_hbm, idx_hbm, out_hbm, idx_v, rows_v, sem0, sem1):
        wid = lax.axis_index("s") * nc + lax.axis_index("c")
        base = wid * b_per_w
        sems = (sem0, sem1)

        def fire(g, buf):
            off = base + g * chunk
            pltpu.sync_copy(idx_hbm.at[pl.ds(row_off + off, chunk)], idx_v.at[buf])
            for t in range(n_slices):
                pltpu.async_copy(
                    tab_hbm.at[idx_v.at[buf], pl.ds(t * 128, 128)],
                    rows_v.at[buf, :, pl.ds(t * 128, 128)],
                    sems[buf],
                )

        def drain_write(g, buf):
            for t in range(n_slices):
                pltpu.make_async_copy(
                    tab_hbm.at[idx_v.at[buf], pl.ds(t * 128, 128)],
                    rows_v.at[buf, :, pl.ds(t * 128, 128)],
                    sems[buf],
                ).wait()
            pltpu.sync_copy(rows_v.at[buf], out_hbm.at[pl.ds(base + g * chunk, chunk)])

        fire(0, 0)

        def body(t, _):
            g = 2 * t

            @pl.when(g + 1 < n_chunks)
            def _():
                fire(g + 1, 1)

            drain_write(g, 0)

            @pl.when(g + 1 < n_chunks)
            def _():
                @pl.when(g + 2 < n_chunks)
                def _():
                    fire(g + 2, 0)

                drain_write(g + 1, 1)

            return 0

        lax.fori_loop(0, (n_chunks + 1) // 2, body, 0)

    return gather


def _mm_compute(a_ref, w_ref, b_ref, o_ref):
    res = (
        jnp.dot(
            a_ref[...].astype(jnp.bfloat16),
            w_ref[...].astype(jnp.bfloat16),
            preferred_element_type=jnp.float32,
        )
        + b_ref[...]
    )
    o_ref[...] = res.reshape(1, -1, D_MODEL)


def _mm_body_first(a_ref, w_ref, b_ref, o_ref):
    _mm_compute(a_ref, w_ref, b_ref, o_ref)


def _mm_body_chained(a_ref, w_ref, b_ref, prev_ref, o_ref):
    del prev_ref
    _mm_compute(a_ref, w_ref, b_ref, o_ref)


def _mm_chunk(emb, wp, b, prev, batch, hist, h_off, h_len):
    in_specs = [
        pl.BlockSpec((batch, DIM_PAD), lambda i: (i, 0)),
        pl.BlockSpec((DIM_PAD, D_MODEL), lambda i: (0, 0)),
        pl.BlockSpec((1, D_MODEL), lambda i: (0, 0)),
    ]
    operands = [emb, wp, b.reshape(1, D_MODEL)]
    aliases = {}
    if prev is not None:
        in_specs.append(pl.BlockSpec(memory_space=pl.ANY))
        operands.append(prev)
        aliases = {3: 0}
    return pl.pallas_call(
        _mm_body_chained if prev is not None else _mm_body_first,
        grid=(h_len,),
        in_specs=in_specs,
        out_specs=pl.BlockSpec((1, batch, D_MODEL), lambda i: (i + h_off, 0, 0)),
        out_shape=jax.ShapeDtypeStruct((hist, batch, D_MODEL), jnp.float32),
        input_output_aliases=aliases,
    )(*operands)


def kernel(x, glove_table, W, b):
    batch, hist = x.shape
    idx = x.T.astype(jnp.int32).reshape(-1)
    tpad = jnp.pad(glove_table, ((0, 0), (0, DIM_PAD - GLOVE_DIM)))
    wp = jnp.pad(W, ((0, DIM_PAD - GLOVE_DIM), (0, 0)))

    assert hist % N_CHUNKS == 0
    h_len = hist // N_CHUNKS
    rows_per_chunk = h_len * batch

    embs = [
        _make_sc_gather(rows_per_chunk, q * rows_per_chunk)(tpad, idx)
        for q in range(N_CHUNKS)
    ]
    out_t = None
    for q in range(N_CHUNKS):
        out_t = _mm_chunk(embs[q], wp, b, out_t, batch, hist, q * h_len, h_len)
    return jnp.transpose(out_t, (1, 0, 2))


# speedup vs baseline: 2.7729x; 2.7729x over previous
import functools

import jax
import jax.numpy as jnp
from jax import lax
from jax.experimental import pallas as pl
from jax.experimental.pallas import tpu as pltpu
from jax.experimental.pallas import tpu_sc as plsc

GLOVE_DIM = 300
D_MODEL = 768
DIM_PAD = 384
N_CHUNKS = 5


def _make_sc_gather(num_rows: int, row_off: int):
    info = plsc.get_sparse_core_info()
    nc, ns = info.num_cores, info.num_subcores
    nw = nc * ns
    assert num_rows % (8 * nw) == 0
    b_per_w = num_rows // nw
    chunk = 80
    assert b_per_w % chunk == 0 and chunk % 8 == 0
    n_chunks = b_per_w // chunk
    n_slices = DIM_PAD // 128

    mesh = plsc.VectorSubcoreMesh(core_axis_name="c", subcore_axis_name="s")

    @functools.partial(
        pl.kernel,
        mesh=mesh,
        out_type=jax.ShapeDtypeStruct((num_rows, DIM_PAD), jnp.float32),
        scratch_types=[
            pltpu.VMEM((2, chunk), jnp.int32),
            pltpu.VMEM((2, chunk, DIM_PAD), jnp.float32),
            pltpu.SemaphoreType.DMA,
            pltpu.SemaphoreType.DMA,
        ],
    )
    def gather(tab_hbm, idx_hbm, out_hbm, idx_v, rows_v, sem0, sem1):
        wid = lax.axis_index("s") * nc + lax.axis_index("c")
        base = wid * b_per_w
        sems = (sem0, sem1)

        def fire(g, buf):
            off = base + g * chunk
            pltpu.sync_copy(idx_hbm.at[pl.ds(row_off + off, chunk)], idx_v.at[buf])
            for t in range(n_slices):
                pltpu.async_copy(
                    tab_hbm.at[idx_v.at[buf], pl.ds(t * 128, 128)],
                    rows_v.at[buf, :, pl.ds(t * 128, 128)],
                    sems[buf],
                )

        def drain_write(g, buf):
            for t in range(n_slices):
                pltpu.make_async_copy(
                    tab_hbm.at[idx_v.at[buf], pl.ds(t * 128, 128)],
                    rows_v.at[buf, :, pl.ds(t * 128, 128)],
                    sems[buf],
                ).wait()
            pltpu.sync_copy(rows_v.at[buf], out_hbm.at[pl.ds(base + g * chunk, chunk)])

        fire(0, 0)

        def body(t, _):
            g = 2 * t

            @pl.when(g + 1 < n_chunks)
            def _():
                fire(g + 1, 1)

            drain_write(g, 0)

            @pl.when(g + 1 < n_chunks)
            def _():
                @pl.when(g + 2 < n_chunks)
                def _():
                    fire(g + 2, 0)

                drain_write(g + 1, 1)

            return 0

        lax.fori_loop(0, (n_chunks + 1) // 2, body, 0)

    return gather


def _repack_body(t_ref, e_ref, o_ref):
    o_ref[...] = lax.dot_general(
        t_ref[...],
        e_ref[...],
        (((0,), (0,)), ((), ())),
        preferred_element_type=jnp.float32,
    )


def _repack_tc(table_t):
    vocab = table_t.shape[1]
    bn = 2048
    eye = jnp.eye(GLOVE_DIM, DIM_PAD, dtype=jnp.float32)
    return pl.pallas_call(
        _repack_body,
        grid=(pl.cdiv(vocab, bn),),
        in_specs=[
            pl.BlockSpec((GLOVE_DIM, bn), lambda i: (0, i)),
            pl.BlockSpec((GLOVE_DIM, DIM_PAD), lambda i: (0, 0)),
        ],
        out_specs=pl.BlockSpec((bn, DIM_PAD), lambda i: (i, 0)),
        out_shape=jax.ShapeDtypeStruct((vocab, DIM_PAD), jnp.float32),
    )(table_t, eye)


def _mm_compute(a_ref, w_ref, b_ref, o_ref):
    res = (
        jnp.dot(
            a_ref[...].astype(jnp.bfloat16),
            w_ref[...].astype(jnp.bfloat16),
            preferred_element_type=jnp.float32,
        )
        + b_ref[...]
    )
    o_ref[...] = res.reshape(1, -1, D_MODEL)


def _mm_body_first(a_ref, w_ref, b_ref, o_ref):
    _mm_compute(a_ref, w_ref, b_ref, o_ref)


def _mm_body_chained(a_ref, w_ref, b_ref, prev_ref, o_ref):
    del prev_ref
    _mm_compute(a_ref, w_ref, b_ref, o_ref)


def _mm_chunk(emb, wp, b, prev, batch, hist, h_off, h_len):
    in_specs = [
        pl.BlockSpec((batch, DIM_PAD), lambda i: (i, 0)),
        pl.BlockSpec((DIM_PAD, D_MODEL), lambda i: (0, 0)),
        pl.BlockSpec((1, D_MODEL), lambda i: (0, 0)),
    ]
    operands = [emb, wp, b.reshape(1, D_MODEL)]
    aliases = {}
    if prev is not None:
        in_specs.append(pl.BlockSpec(memory_space=pl.ANY))
        operands.append(prev)
        aliases = {3: 0}
    return pl.pallas_call(
        _mm_body_chained if prev is not None else _mm_body_first,
        grid=(h_len,),
        in_specs=in_specs,
        out_specs=pl.BlockSpec((1, batch, D_MODEL), lambda i: (i + h_off, 0, 0)),
        out_shape=jax.ShapeDtypeStruct((hist, batch, D_MODEL), jnp.float32),
        input_output_aliases=aliases,
    )(*operands)


def kernel(x, glove_table, W, b):
    batch, hist = x.shape
    idx = x.T.astype(jnp.int32).reshape(-1)
    tpad = _repack_tc(glove_table.T)
    wp = jnp.pad(W, ((0, DIM_PAD - GLOVE_DIM), (0, 0)))

    assert hist % N_CHUNKS == 0
    h_len = hist // N_CHUNKS
    rows_per_chunk = h_len * batch

    embs = [
        _make_sc_gather(rows_per_chunk, q * rows_per_chunk)(tpad, idx)
        for q in range(N_CHUNKS)
    ]
    out_t = None
    for q in range(N_CHUNKS):
        out_t = _mm_chunk(embs[q], wp, b, out_t, batch, hist, q * h_len, h_len)
    return jnp.transpose(out_t, (1, 0, 2))


# speedup vs baseline: 2.9566x; 1.0662x over previous
import functools

import jax
import jax.numpy as jnp
from jax import lax
from jax.experimental import pallas as pl
from jax.experimental.pallas import tpu as pltpu
from jax.experimental.pallas import tpu_sc as plsc

GLOVE_DIM = 300
D_MODEL = 768
PACK_W = 256
BF_W = 2 * PACK_W
N_CHUNKS = 5


def _make_sc_gather(num_rows: int, row_off: int):
    info = plsc.get_sparse_core_info()
    nc, ns = info.num_cores, info.num_subcores
    nw = nc * ns
    assert num_rows % (8 * nw) == 0
    b_per_w = num_rows // nw
    chunk = 80
    assert b_per_w % chunk == 0 and chunk % 8 == 0
    n_chunks = b_per_w // chunk
    n_slices = PACK_W // 128

    mesh = plsc.VectorSubcoreMesh(core_axis_name="c", subcore_axis_name="s")

    @functools.partial(
        pl.kernel,
        mesh=mesh,
        out_type=jax.ShapeDtypeStruct((num_rows, PACK_W), jnp.int32),
        scratch_types=[
            pltpu.VMEM((2, chunk), jnp.int32),
            pltpu.VMEM((2, chunk, PACK_W), jnp.int32),
            pltpu.SemaphoreType.DMA,
            pltpu.SemaphoreType.DMA,
        ],
    )
    def gather(tab_hbm, idx_hbm, out_hbm, idx_v, rows_v, sem0, sem1):
        wid = lax.axis_index("s") * nc + lax.axis_index("c")
        base = wid * b_per_w
        sems = (sem0, sem1)

        def fire(g, buf):
            off = base + g * chunk
            pltpu.sync_copy(idx_hbm.at[pl.ds(row_off + off, chunk)], idx_v.at[buf])
            for t in range(n_slices):
                pltpu.async_copy(
                    tab_hbm.at[idx_v.at[buf], pl.ds(t * 128, 128)],
                    rows_v.at[buf, :, pl.ds(t * 128, 128)],
                    sems[buf],
                )

        def drain_write(g, buf):
            for t in range(n_slices):
                pltpu.make_async_copy(
                    tab_hbm.at[idx_v.at[buf], pl.ds(t * 128, 128)],
                    rows_v.at[buf, :, pl.ds(t * 128, 128)],
                    sems[buf],
                ).wait()
            pltpu.sync_copy(rows_v.at[buf], out_hbm.at[pl.ds(base + g * chunk, chunk)])

        fire(0, 0)

        def body(t, _):
            g = 2 * t

            @pl.when(g + 1 < n_chunks)
            def _():
                fire(g + 1, 1)

            drain_write(g, 0)

            @pl.when(g + 1 < n_chunks)
            def _():
                @pl.when(g + 2 < n_chunks)
                def _():
                    fire(g + 2, 0)

                drain_write(g + 1, 1)

            return 0

        lax.fori_loop(0, (n_chunks + 1) // 2, body, 0)

    return gather


def _repack_body(t_ref, e_ref, o_ref):
    res = lax.dot_general(
        t_ref[...],
        e_ref[...],
        (((0,), (0,)), ((), ())),
        preferred_element_type=jnp.float32,
    )
    u = lax.bitcast_convert_type(res, jnp.uint32)
    rne = (u + jnp.uint32(0x7FFF) + ((u >> 16) & jnp.uint32(1))) >> 16
    lo = rne[:, :PACK_W]
    hi = rne[:, PACK_W:]
    o_ref[...] = lax.bitcast_convert_type(lo | (hi << 16), jnp.int32)


def _repack_tc(table_t):
    vocab = table_t.shape[1]
    bn = 2048
    eye = jnp.eye(GLOVE_DIM, BF_W, dtype=jnp.float32)
    return pl.pallas_call(
        _repack_body,
        grid=(pl.cdiv(vocab, bn),),
        in_specs=[
            pl.BlockSpec((GLOVE_DIM, bn), lambda i: (0, i)),
            pl.BlockSpec((GLOVE_DIM, BF_W), lambda i: (0, 0)),
        ],
        out_specs=pl.BlockSpec((bn, PACK_W), lambda i: (i, 0)),
        out_shape=jax.ShapeDtypeStruct((vocab, PACK_W), jnp.int32),
    )(table_t, eye)


def _mm_compute(a_ref, w_ref, b_ref, o_ref):
    au = lax.bitcast_convert_type(a_ref[...], jnp.uint32)
    a_lo = lax.bitcast_convert_type(au << 16, jnp.float32)
    a_hi = lax.bitcast_convert_type(au & jnp.uint32(0xFFFF0000), jnp.float32)
    w = w_ref[...].astype(jnp.bfloat16)
    res = jnp.dot(
        a_lo.astype(jnp.bfloat16),
        w[:PACK_W],
        preferred_element_type=jnp.float32,
    )
    res += jnp.dot(
        a_hi.astype(jnp.bfloat16),
        w[PACK_W:],
        preferred_element_type=jnp.float32,
    )
    o_ref[...] = (res + b_ref[...]).reshape(1, -1, D_MODEL)


def _mm_body_first(a_ref, w_ref, b_ref, o_ref):
    _mm_compute(a_ref, w_ref, b_ref, o_ref)


def _mm_body_chained(a_ref, w_ref, b_ref, prev_ref, o_ref):
    del prev_ref
    _mm_compute(a_ref, w_ref, b_ref, o_ref)


def _mm_chunk(emb, wp, b, prev, batch, hist, h_off, h_len):
    in_specs = [
        pl.BlockSpec((batch, PACK_W), lambda i: (i, 0)),
        pl.BlockSpec((BF_W, D_MODEL), lambda i: (0, 0)),
        pl.BlockSpec((1, D_MODEL), lambda i: (0, 0)),
    ]
    operands = [emb, wp, b.reshape(1, D_MODEL)]
    aliases = {}
    if prev is not None:
        in_specs.append(pl.BlockSpec(memory_space=pl.ANY))
        operands.append(prev)
        aliases = {3: 0}
    return pl.pallas_call(
        _mm_body_chained if prev is not None else _mm_body_first,
        grid=(h_len,),
        in_specs=in_specs,
        out_specs=pl.BlockSpec((1, batch, D_MODEL), lambda i: (i + h_off, 0, 0)),
        out_shape=jax.ShapeDtypeStruct((hist, batch, D_MODEL), jnp.float32),
        input_output_aliases=aliases,
    )(*operands)


def kernel(x, glove_table, W, b):
    batch, hist = x.shape
    idx = x.T.astype(jnp.int32).reshape(-1)
    tpad = _repack_tc(glove_table.T)
    wp = jnp.pad(W, ((0, BF_W - GLOVE_DIM), (0, 0)))

    assert hist % N_CHUNKS == 0
    h_len = hist // N_CHUNKS
    rows_per_chunk = h_len * batch

    embs = [
        _make_sc_gather(rows_per_chunk, q * rows_per_chunk)(tpad, idx)
        for q in range(N_CHUNKS)
    ]
    out_t = None
    for q in range(N_CHUNKS):
        out_t = _mm_chunk(embs[q], wp, b, out_t, batch, hist, q * h_len, h_len)
    return jnp.transpose(out_t, (1, 0, 2))


# speedup vs baseline: 3.1981x; 1.0817x over previous
import functools

import jax
import jax.numpy as jnp
from jax import lax
from jax.experimental import pallas as pl
from jax.experimental.pallas import tpu as pltpu
from jax.experimental.pallas import tpu_sc as plsc

GLOVE_DIM = 300
D_MODEL = 768
PACK_W = 256
BF_W = 2 * PACK_W
N_CHUNKS = 5


def _make_sc_gather(num_rows: int, row_off: int):
    info = plsc.get_sparse_core_info()
    nc, ns = info.num_cores, info.num_subcores
    nw = nc * ns
    assert num_rows % (8 * nw) == 0
    b_per_w = num_rows // nw
    chunk = 80
    assert b_per_w % chunk == 0 and chunk % 8 == 0
    n_chunks = b_per_w // chunk
    n_slices = PACK_W // 128

    mesh = plsc.VectorSubcoreMesh(core_axis_name="c", subcore_axis_name="s")

    @functools.partial(
        pl.kernel,
        mesh=mesh,
        out_type=jax.ShapeDtypeStruct((num_rows, PACK_W), jnp.int32),
        scratch_types=[
            pltpu.VMEM((2, chunk), jnp.int32),
            pltpu.VMEM((2, chunk, PACK_W), jnp.int32),
            pltpu.SemaphoreType.DMA,
            pltpu.SemaphoreType.DMA,
        ],
    )
    def gather(tab_hbm, idx_hbm, out_hbm, idx_v, rows_v, sem0, sem1):
        wid = lax.axis_index("s") * nc + lax.axis_index("c")
        base = wid * b_per_w
        sems = (sem0, sem1)

        def fire(g, buf):
            off = base + g * chunk
            pltpu.sync_copy(idx_hbm.at[pl.ds(row_off + off, chunk)], idx_v.at[buf])
            for t in range(n_slices):
                pltpu.async_copy(
                    tab_hbm.at[idx_v.at[buf], pl.ds(t * 128, 128)],
                    rows_v.at[buf, :, pl.ds(t * 128, 128)],
                    sems[buf],
                )

        def drain_write(g, buf):
            for t in range(n_slices):
                pltpu.make_async_copy(
                    tab_hbm.at[idx_v.at[buf], pl.ds(t * 128, 128)],
                    rows_v.at[buf, :, pl.ds(t * 128, 128)],
                    sems[buf],
                ).wait()
            pltpu.sync_copy(rows_v.at[buf], out_hbm.at[pl.ds(base + g * chunk, chunk)])

        fire(0, 0)

        def body(t, _):
            g = 2 * t

            @pl.when(g + 1 < n_chunks)
            def _():
                fire(g + 1, 1)

            drain_write(g, 0)

            @pl.when(g + 1 < n_chunks)
            def _():
                @pl.when(g + 2 < n_chunks)
                def _():
                    fire(g + 2, 0)

                drain_write(g + 1, 1)

            return 0

        lax.fori_loop(0, (n_chunks + 1) // 2, body, 0)

    return gather


def _repack_body(t_ref, o_ref):
    res = jnp.swapaxes(t_ref[...], 0, 1)
    u = lax.bitcast_convert_type(res, jnp.uint32)
    rne = (u + jnp.uint32(0x7FFF) + ((u >> 16) & jnp.uint32(1))) >> 16
    lo = rne[:, :PACK_W]
    hi = jnp.concatenate(
        [
            rne[:, PACK_W:],
            jnp.zeros((res.shape[0], 2 * PACK_W - GLOVE_DIM), jnp.uint32),
        ],
        axis=1,
    )
    o_ref[...] = lax.bitcast_convert_type(lo | (hi << 16), jnp.int32)


def _repack_tc(table_t):
    vocab = table_t.shape[1]
    bn = 2048
    return pl.pallas_call(
        _repack_body,
        grid=(pl.cdiv(vocab, bn),),
        in_specs=[pl.BlockSpec((GLOVE_DIM, bn), lambda i: (0, i))],
        out_specs=pl.BlockSpec((bn, PACK_W), lambda i: (i, 0)),
        out_shape=jax.ShapeDtypeStruct((vocab, PACK_W), jnp.int32),
    )(table_t)


def _mm_compute(a_ref, w_ref, b_ref, o_ref):
    au = lax.bitcast_convert_type(a_ref[...], jnp.uint32)
    a_lo = lax.bitcast_convert_type(au << 16, jnp.float32)
    a_hi = lax.bitcast_convert_type(au & jnp.uint32(0xFFFF0000), jnp.float32)
    w = w_ref[...].astype(jnp.bfloat16)
    res = jnp.dot(
        a_lo.astype(jnp.bfloat16),
        w[:PACK_W],
        preferred_element_type=jnp.float32,
    )
    res += jnp.dot(
        a_hi.astype(jnp.bfloat16),
        w[PACK_W:],
        preferred_element_type=jnp.float32,
    )
    o_ref[...] = (res + b_ref[...]).reshape(1, -1, D_MODEL)


def _mm_body_first(a_ref, w_ref, b_ref, o_ref):
    _mm_compute(a_ref, w_ref, b_ref, o_ref)


def _mm_body_chained(a_ref, w_ref, b_ref, prev_ref, o_ref):
    del prev_ref
    _mm_compute(a_ref, w_ref, b_ref, o_ref)


def _mm_chunk(emb, wp, b, prev, batch, hist, h_off, h_len):
    in_specs = [
        pl.BlockSpec((batch, PACK_W), lambda i: (i, 0)),
        pl.BlockSpec((BF_W, D_MODEL), lambda i: (0, 0)),
        pl.BlockSpec((1, D_MODEL), lambda i: (0, 0)),
    ]
    operands = [emb, wp, b.reshape(1, D_MODEL)]
    aliases = {}
    if prev is not None:
        in_specs.append(pl.BlockSpec(memory_space=pl.ANY))
        operands.append(prev)
        aliases = {3: 0}
    return pl.pallas_call(
        _mm_body_chained if prev is not None else _mm_body_first,
        grid=(h_len,),
        in_specs=in_specs,
        out_specs=pl.BlockSpec((1, batch, D_MODEL), lambda i: (i + h_off, 0, 0)),
        out_shape=jax.ShapeDtypeStruct((hist, batch, D_MODEL), jnp.float32),
        input_output_aliases=aliases,
    )(*operands)


def kernel(x, glove_table, W, b):
    batch, hist = x.shape
    idx = x.T.astype(jnp.int32).reshape(-1)
    tpad = _repack_tc(glove_table.T)
    wp = jnp.pad(W, ((0, BF_W - GLOVE_DIM), (0, 0)))

    assert hist % N_CHUNKS == 0
    h_len = hist // N_CHUNKS
    rows_per_chunk = h_len * batch

    embs = [
        _make_sc_gather(rows_per_chunk, q * rows_per_chunk)(tpad, idx)
        for q in range(N_CHUNKS)
    ]
    out_t = None
    for q in range(N_CHUNKS):
        out_t = _mm_chunk(embs[q], wp, b, out_t, batch, hist, q * h_len, h_len)
    return jnp.transpose(out_t, (1, 0, 2))
